# baseline (speedup 1.0000x reference)
import functools

import jax
import jax.numpy as jnp
from jax import lax
from jax.experimental import pallas as pl
from jax.experimental.pallas import tpu as pltpu
from jax.experimental.pallas import tpu_sc as plsc

E = 320000
D = 128


_BC = 12800


def _prep_body(attrs_t_ref, w0_ref, w1_ref, w2_ref, c_ref, t_ref):
    a0 = attrs_t_ref[0:1, :]
    a1 = attrs_t_ref[1:2, :]
    a2 = attrs_t_ref[2:3, :]
    c_ref[:] = a0 * 12 + a1 * 2 + a2

    @pl.when(pl.program_id(0) == 0)
    def _():
        i = lax.broadcasted_iota(jnp.int32, (60, 1), 0)
        i0, i1, i2 = i // 12, (i // 2) % 6, i % 2
        oh0 = (lax.broadcasted_iota(jnp.int32, (60, 8), 1) == i0).astype(jnp.float32)
        oh1 = (lax.broadcasted_iota(jnp.int32, (60, 8), 1) == i1).astype(jnp.float32)
        oh2 = (lax.broadcasted_iota(jnp.int32, (60, 8), 1) == i2).astype(jnp.float32)
        w0p = jnp.concatenate([w0_ref[:], jnp.zeros((3, D), jnp.float32)], axis=0)
        w1p = jnp.concatenate([w1_ref[:], jnp.zeros((2, D), jnp.float32)], axis=0)
        w2p = jnp.concatenate([w2_ref[:], jnp.zeros((6, D), jnp.float32)], axis=0)
        t_ref[:] = (
            jnp.dot(oh0, w0p, preferred_element_type=jnp.float32)
            + jnp.dot(oh1, w1p, preferred_element_type=jnp.float32)
            + jnp.dot(oh2, w2p, preferred_element_type=jnp.float32)
        )


def _prep(attrs_t, W0, W1, W2):
    return pl.pallas_call(
        _prep_body,
        grid=(E // _BC,),
        in_specs=[
            pl.BlockSpec((3, _BC), lambda i: (0, i)),
            pl.BlockSpec((5, D), lambda i: (0, 0)),
            pl.BlockSpec((6, D), lambda i: (0, 0)),
            pl.BlockSpec((2, D), lambda i: (0, 0)),
        ],
        out_specs=[
            pl.BlockSpec((1, _BC), lambda i: (0, i)),
            pl.BlockSpec((60, D), lambda i: (0, 0)),
        ],
        out_shape=[
            jax.ShapeDtypeStruct((1, E), jnp.int32),
            jax.ShapeDtypeStruct((60, D), jnp.float32),
        ],
    )(attrs_t, W0, W1, W2)


_NW = 32
_RPW = E // _NW
_R = 400
_DV = D // 16


def _sc_body(msg_hbm, c_hbm, t_hbm, out_hbm, c_v, m_v, e_v, sem1, sem2):
    core = lax.axis_index("c")
    sub = lax.axis_index("s")
    wid = sub * 2 + core
    base = wid * _RPW

    def chunk(i, _):
        rb = base + i * _R
        pltpu.sync_copy(c_hbm.at[pl.ds(rb, _R)], c_v)
        cp_m = pltpu.async_copy(msg_hbm.at[pl.ds(rb, _R)], m_v, sem1)
        cp_e = pltpu.async_copy(t_hbm.at[c_v], e_v, sem2)
        cp_m.wait()
        cp_e.wait()

        def row(r, _):
            for j in range(_DV):
                s = pl.ds(j * 16, 16)
                m_v[r, s] = jnp.maximum(m_v[r, s] + e_v[r, s], 0.0)
            return 0

        lax.fori_loop(0, _R, row, 0)
        pltpu.sync_copy(m_v, out_hbm.at[pl.ds(rb, _R)])
        return 0

    lax.fori_loop(0, _RPW // _R, chunk, 0)


@functools.partial(
    pl.kernel,
    mesh=plsc.VectorSubcoreMesh(core_axis_name="c", subcore_axis_name="s"),
    out_type=jax.ShapeDtypeStruct((E, D), jnp.float32),
    scratch_types=[
        pltpu.VMEM((_R,), jnp.int32),
        pltpu.VMEM((_R, D), jnp.float32),
        pltpu.VMEM((_R, D), jnp.float32),
        pltpu.SemaphoreType.DMA,
        pltpu.SemaphoreType.DMA,
    ],
    compiler_params=pltpu.CompilerParams(use_tc_tiling_on_sc=False),
)
def _sc_main(msg_hbm, c_hbm, t_hbm, out_hbm, c_v, m_v, e_v, sem1, sem2):
    _sc_body(msg_hbm, c_hbm, t_hbm, out_hbm, c_v, m_v, e_v, sem1, sem2)


@jax.jit
def kernel(message, attrs, W0, W1, W2):
    attrs_t = attrs.astype(jnp.int32).T
    c2d, tcomb = _prep(attrs_t, W0, W1, W2)
    return _sc_main(message, c2d.reshape(E), tcomb)


# speedup vs baseline: 4.9740x; 4.9740x over previous
import functools

import jax
import jax.numpy as jnp
from jax import lax
from jax.experimental import pallas as pl
from jax.experimental.pallas import tpu as pltpu
from jax.experimental.pallas import tpu_sc as plsc

E = 320000
D = 128

_E_SC = 128000
_E_TC = E - _E_SC


_BC = 12800


def _prep_body(attrs_t_ref, w0_ref, w1_ref, w2_ref, c_ref, t_ref):
    a0 = attrs_t_ref[0:1, :]
    a1 = attrs_t_ref[1:2, :]
    a2 = attrs_t_ref[2:3, :]
    c_ref[:] = a0 * 12 + a1 * 2 + a2

    @pl.when(pl.program_id(0) == 0)
    def _():
        i = lax.broadcasted_iota(jnp.int32, (60, 1), 0)
        i0, i1, i2 = i // 12, (i // 2) % 6, i % 2
        oh0 = (lax.broadcasted_iota(jnp.int32, (60, 8), 1) == i0).astype(jnp.float32)
        oh1 = (lax.broadcasted_iota(jnp.int32, (60, 8), 1) == i1).astype(jnp.float32)
        oh2 = (lax.broadcasted_iota(jnp.int32, (60, 8), 1) == i2).astype(jnp.float32)
        w0p = jnp.concatenate([w0_ref[:], jnp.zeros((3, D), jnp.float32)], axis=0)
        w1p = jnp.concatenate([w1_ref[:], jnp.zeros((2, D), jnp.float32)], axis=0)
        w2p = jnp.concatenate([w2_ref[:], jnp.zeros((6, D), jnp.float32)], axis=0)
        t_ref[:] = (
            jnp.dot(oh0, w0p, preferred_element_type=jnp.float32)
            + jnp.dot(oh1, w1p, preferred_element_type=jnp.float32)
            + jnp.dot(oh2, w2p, preferred_element_type=jnp.float32)
        )


def _prep(attrs_t, W0, W1, W2):
    return pl.pallas_call(
        _prep_body,
        grid=(_E_SC // _BC,),
        in_specs=[
            pl.BlockSpec((3, _BC), lambda i: (0, i)),
            pl.BlockSpec((5, D), lambda i: (0, 0)),
            pl.BlockSpec((6, D), lambda i: (0, 0)),
            pl.BlockSpec((2, D), lambda i: (0, 0)),
        ],
        out_specs=[
            pl.BlockSpec((1, _BC), lambda i: (0, i)),
            pl.BlockSpec((60, D), lambda i: (0, 0)),
        ],
        out_shape=[
            jax.ShapeDtypeStruct((1, _E_SC), jnp.int32),
            jax.ShapeDtypeStruct((60, D), jnp.float32),
        ],
    )(attrs_t, W0, W1, W2)


_NW = 32
_RPW = _E_SC // _NW
_R = 400
_DV = D // 16


def _sc_body(msg_hbm, c_hbm, t_hbm, out_hbm, t_v, c_v, m_v, o_v):
    core = lax.axis_index("c")
    sub = lax.axis_index("s")
    wid = sub * 2 + core
    base = wid * _RPW
    pltpu.sync_copy(t_hbm, t_v)

    def chunk(i, _):
        rb = base + i * _R
        pltpu.sync_copy(c_hbm.at[pl.ds(rb, _R)], c_v)
        pltpu.sync_copy(msg_hbm.at[pl.ds(rb * _DV, _R * _DV)], m_v)

        def group(g, _):
            cg = c_v[pl.ds(g * 16, 16)] * _DV
            for l in range(16):
                trow = cg[l]
                mrow = (g * 16 + l) * _DV
                for j in range(_DV):
                    o_v[mrow + j] = jnp.maximum(m_v[mrow + j] + t_v[trow + j], 0.0)
            return 0

        lax.fori_loop(0, _R // 16, group, 0)
        pltpu.sync_copy(o_v, out_hbm.at[pl.ds(rb * _DV, _R * _DV)])
        return 0

    lax.fori_loop(0, _RPW // _R, chunk, 0)


@functools.partial(
    pl.kernel,
    mesh=plsc.VectorSubcoreMesh(core_axis_name="c", subcore_axis_name="s"),
    out_type=jax.ShapeDtypeStruct((_E_SC * _DV, 16), jnp.float32),
    scratch_types=[
        pltpu.VMEM((60 * _DV, 16), jnp.float32),
        pltpu.VMEM((_R,), jnp.int32),
        pltpu.VMEM((_R * _DV, 16), jnp.float32),
        pltpu.VMEM((_R * _DV, 16), jnp.float32),
    ],
    compiler_params=pltpu.CompilerParams(use_tc_tiling_on_sc=False),
)
def _sc_main(msg_hbm, c_hbm, t_hbm, out_hbm, t_v, c_v, m_v, o_v):
    _sc_body(msg_hbm, c_hbm, t_hbm, out_hbm, t_v, c_v, m_v, o_v)


_B = 1600


def _tc_body(attrs_ref, msg_ref, w0_ref, w1_ref, w2_ref, out_ref):
    a0 = attrs_ref[:, 0:1]
    a1 = attrs_ref[:, 1:2]
    a2 = attrs_ref[:, 2:3]
    iota = lax.broadcasted_iota(jnp.int32, (attrs_ref.shape[0], 16), 1)
    oh = ((iota == a0) | (iota == a1 + 5) | (iota == a2 + 11)).astype(jnp.float32)
    wcat = jnp.concatenate(
        [w0_ref[:], w1_ref[:], w2_ref[:], jnp.zeros((3, D), jnp.float32)], axis=0
    )
    emb = jnp.dot(oh, wcat, preferred_element_type=jnp.float32)
    out_ref[:] = jnp.maximum(msg_ref[:] + emb, 0.0)


def _tc_main(attrs, message, W0, W1, W2):
    nblk = _E_TC // _B
    off = _E_SC // _B
    return pl.pallas_call(
        _tc_body,
        grid=(nblk,),
        in_specs=[
            pl.BlockSpec((_B, 3), lambda i: (i + off, 0)),
            pl.BlockSpec((_B, D), lambda i: (i + off, 0)),
            pl.BlockSpec((5, D), lambda i: (0, 0)),
            pl.BlockSpec((6, D), lambda i: (0, 0)),
            pl.BlockSpec((2, D), lambda i: (0, 0)),
        ],
        out_specs=pl.BlockSpec((_B, D), lambda i: (i, 0)),
        out_shape=jax.ShapeDtypeStruct((_E_TC, D), jnp.float32),
    )(attrs, message, W0, W1, W2)


@jax.jit
def kernel(message, attrs, W0, W1, W2):
    attrs = attrs.astype(jnp.int32)
    attrs_t = attrs[:_E_SC].T
    c2d, tcomb = _prep(attrs_t, W0, W1, W2)
    msg2 = message.reshape(E * _DV, 16)
    t2 = tcomb.reshape(60 * _DV, 16)
    out_sc = _sc_main(msg2, c2d.reshape(_E_SC), t2)
    out_tc = _tc_main(attrs, message, W0, W1, W2)
    return jnp.concatenate([out_sc.reshape(_E_SC, D), out_tc], axis=0)
